# baseline (speedup 1.0000x reference)
import functools

import jax
import jax.numpy as jnp
from jax import lax
from jax.experimental import pallas as pl
from jax.experimental.pallas import tpu as pltpu

_B = 16
_HID = 32
_NV = 16
_NRBF = 16
_DMAX = 10.0
_NORM = 100.0
_F32 = jnp.float32
_SEGPAD = 8000000.0


def _silu(x):
    return x * jax.nn.sigmoid(x)


def _dot(a, b):
    return jnp.dot(a, b, preferred_element_type=_F32)


def _embed_body(t_ref, h_ref, lig_ref, xs_ref, ligwt_ref, ligb_ref,
                hwt_ref, hb_ref, nodewt_ref, nodeb_ref,
                wsi1_ref, wsj1_ref, bs1_ref,
                w1_ref, s_ref):
    B = h_ref.shape[0]
    tcol = jnp.broadcast_to(t_ref[...], (B, 1))
    lig_in = jnp.concatenate([lig_ref[...], tcol], axis=1)
    lig_e = _silu(_dot(lig_in, ligwt_ref[...]) + ligb_ref[...]) * (1.0 / 0.6)
    h_e = _silu(_dot(h_ref[...], hwt_ref[...]) + hb_ref[...]) * (1.0 / 0.6)
    s0 = jnp.concatenate([h_e, lig_e], axis=1)
    s = _silu(_dot(s0, nodewt_ref[...]) + nodeb_ref[...])
    asi = _dot(s, wsi1_ref[...]) + bs1_ref[...]
    asj = _dot(s, wsj1_ref[...])
    w1_ref[...] = jnp.concatenate([asi, asj, xs_ref[...]], axis=1)
    s_ref[...] = s


def _layer_body(first, last, B, *args):
    if first:
        (lo_ref, hi_ref, w1_ref, xr_ref, seld_ref, sel33_ref,
         sel48_ref, s_ref, *rest) = args
        w2_ref = w3_ref = sv_ref = None
    else:
        (lo_ref, hi_ref, w1_ref, xr_ref, seld_ref, sel33_ref,
         sel48_ref, w2_ref, w3_ref, sv_ref, *rest) = args
    (wh399_ref, wsr_ref, wsv_ref, bg_ref, wgt_ref, wu48_ref,
     whf_ref, wsfs_ref, wsfv_ref, bsf_ref, wuf_ref, wgft_ref, bgf_ref,
     *tail) = rest
    if last:
        (who_ref, wsos_ref, wsov_ref, bso_ref, wuo_ref, wgot_ref, bgo_ref,
         outwt_ref, outb_ref, out_ref) = tail
    else:
        (wsi2_ref, wsj2_ref, bs2_ref, whi2_ref, whj2_ref,
         w1o_ref, w2o_ref, w3o_ref, w4o_ref) = tail

    i = pl.program_id(0)
    r0 = i * B
    P = B * B

    w1r = w1_ref[pl.ds(r0, B), :]
    asi3 = w1r[:, 0:32].reshape(B, 1, _HID)
    segr3 = w1r[:, 67:68].reshape(B, 1, 1)
    xr99 = xr_ref[pl.ds(r0, B), :].reshape(B, 1, 99)
    ri = lax.broadcasted_iota(jnp.int32, (B, 1, 1), 0) + r0
    if not first:
        aiwh3 = w2_ref[pl.ds(r0, B), :].reshape(B, 1, 99)

    wh399 = wh399_ref[...].reshape(1, 1, 99)
    wsr = wsr_ref[...]
    wsv = wsv_ref[...]
    bg = bg_ref[...]
    wgt = wgt_ref[...]
    wu48 = wu48_ref[...]
    seld = seld_ref[...]
    sel33 = sel33_ref[...]
    sel48 = sel48_ref[...]
    mu3 = (lax.broadcasted_iota(jnp.int32, (1, 1, _NRBF), 2).astype(_F32)
           * (_DMAX / (_NRBF - 1)))
    inv_sig = _NRBF / _DMAX

    def pair_block(j, carry):
        s_acc, v_acc = carry
        c0 = j * B
        w1c = w1_ref[pl.ds(c0, B), :]
        asj3 = w1c[:, 32:64].reshape(1, B, _HID)
        segc3 = w1c[:, 67:68].reshape(1, B, 1)
        rel99 = xr99 - xr_ref[pl.ds(c0, B), :].reshape(1, B, 99)
        d2p = _dot((rel99 * rel99).reshape(P, 99), seld)
        tp = d2p + 1e-8
        invp = lax.rsqrt(tp)
        d3 = (tp * invp).reshape(B, B, 1)
        invd3 = invp.reshape(B, B, 1)
        rbf = jnp.exp(-(((d3 - mu3) * inv_sig) ** 2))
        ci = lax.broadcasted_iota(jnp.int32, (1, B, 1), 1) + c0
        mask = jnp.logical_and(segr3 == segc3, ri != ci)
        mf1 = mask.astype(_F32).reshape(P, 1)
        if first:
            vh = (rel99 * invd3) * wh399
        else:
            vh = (aiwh3 + w3_ref[pl.ds(c0, B), :].reshape(1, B, 99)
                  + (rel99 * invd3) * wh399)
        sq = vh * vh
        vn = jnp.sqrt(_dot(sq.reshape(P, 99), sel33) + 1e-8)
        sm = ((asi3 + asj3).reshape(P, _HID)
              + _dot(rbf.reshape(P, _NRBF), wsr)
              + _dot(vn, wsv))
        sma = sm * jax.nn.sigmoid(sm)
        gate = jax.nn.sigmoid(_dot(sma, wgt) + bg)
        vu = _dot(vh.reshape(P, 99), wu48)
        s_add = (sma * mf1).reshape(B, B, _HID).sum(axis=1)
        gm48 = _dot(gate * mf1, sel48)
        v_add = (vu * gm48).reshape(B, B, 48).sum(axis=1)
        return (s_acc + s_add, v_acc + v_add)

    init = (jnp.zeros((B, _HID), _F32), jnp.zeros((B, 48), _F32))
    jlo = lo_ref[i]
    jhi = hi_ref[i]
    s_acc, v_acc = lax.fori_loop(jlo, jhi + 1, pair_block, init)

    if first:
        s_r = s_ref[pl.ds(r0, B), :]
        v1c = [v_acc[:, 16 * c:16 * c + 16] * (1.0 / _NORM) for c in range(3)]
    else:
        svr = sv_ref[pl.ds(r0, B), :]
        s_r = svr[:, 0:32]
        v1c = [svr[:, 32 + 16 * c:48 + 16 * c]
               + v_acc[:, 16 * c:16 * c + 16] * (1.0 / _NORM)
               for c in range(3)]
    s1 = s_r + s_acc * (1.0 / _NORM)

    vhf = [_dot(v1c[c], whf_ref[...]) for c in range(3)]
    vnf = jnp.sqrt(vhf[0] * vhf[0] + vhf[1] * vhf[1] + vhf[2] * vhf[2] + 1e-8)
    smf = _dot(s1, wsfs_ref[...]) + _dot(vnf, wsfv_ref[...]) + bsf_ref[...]
    smfa = smf * jax.nn.sigmoid(smf)
    gatef = jax.nn.sigmoid(_dot(smfa, wgft_ref[...]) + bgf_ref[...])
    s2 = s1 + smfa
    v2c = [v1c[c] + _dot(vhf[c], wuf_ref[...]) * gatef for c in range(3)]

    if not last:
        asi2 = _dot(s2, wsi2_ref[...]) + bs2_ref[...]
        asj2 = _dot(s2, wsj2_ref[...])
        w1o_ref[...] = jnp.concatenate([asi2, asj2, w1r[:, 64:68]], axis=1)
        w2o_ref[...] = jnp.concatenate(
            [_dot(v2c[c], whi2_ref[...]) for c in range(3)], axis=1)
        w3o_ref[...] = jnp.concatenate(
            [_dot(v2c[c], whj2_ref[...]) for c in range(3)], axis=1)
        w4o_ref[...] = jnp.concatenate([s2] + v2c, axis=1)
    else:
        vho = [_dot(v2c[c], who_ref[...]) for c in range(3)]
        vno = jnp.sqrt(vho[0] * vho[0] + vho[1] * vho[1] + vho[2] * vho[2]
                       + 1e-8)
        smo = _dot(s2, wsos_ref[...]) + _dot(vno, wsov_ref[...]) + bso_ref[...]
        smoa = smo * jax.nn.sigmoid(smo)
        gateo = jax.nn.sigmoid(_dot(smoa, wgot_ref[...]) + bgo_ref[...])
        vel = [_dot(vho[c], wuo_ref[...]) * gateo for c in range(3)]
        hf = _dot(smo, outwt_ref[...]) + outb_ref[...]
        out_ref[...] = jnp.concatenate(vel + [hf], axis=1)


def _full(a):
    nd = a.ndim
    return pl.BlockSpec(a.shape, lambda i, _nd=nd: (0,) * _nd)


def _smem():
    return pl.BlockSpec(memory_space=pltpu.SMEM)


def kernel(t, xh, batch_seg, ligand_site, params):
    N = xh.shape[0]
    B = _B
    NB = -(-N // B)
    NPAD = NB * B
    padn = NPAD - N

    x = jnp.pad(xh[:, :3].astype(_F32), ((0, padn), (0, 0)))
    h = jnp.pad(xh[:, 3:].astype(_F32), ((0, padn), (0, 0)))
    lig = jnp.pad(ligand_site.astype(_F32), ((0, padn), (0, 0)))
    seg = batch_seg.astype(jnp.int32)
    segp = jnp.pad(seg, (0, padn), constant_values=jnp.int32(int(_SEGPAD)))
    segf = segp.astype(_F32).reshape(NPAD, 1)
    xs = jnp.concatenate([x, segf], axis=1)
    xr99 = jnp.repeat(x, 33, axis=1)
    seld = jnp.zeros((99, 1), _F32).at[0, 0].set(1.0)
    seld = seld.at[33, 0].set(1.0).at[66, 0].set(1.0)
    sel33 = jnp.tile(jnp.eye(33, dtype=_F32), (3, 1))
    sel48 = jnp.tile(jnp.eye(16, dtype=_F32), (1, 3))
    blocks = segp.reshape(NB, B)
    lo_blk = (jnp.searchsorted(segp, blocks[:, 0], side="left") // B
              ).astype(jnp.int32)
    hi_blk = ((jnp.searchsorted(segp, blocks[:, -1], side="right") - 1) // B
              ).astype(jnp.int32)
    t2 = t.reshape(1, 1).astype(_F32)

    p = params

    def msg_prep(mp):
        wst = mp["Ws"].T
        wu = mp["Wu"]
        wu48 = jnp.zeros((99, 48), _F32)
        for c in range(3):
            wu48 = wu48.at[33 * c:33 * c + 33, 16 * c:16 * c + 16].set(wu)
        return dict(whi=mp["Wh"][16:32, :], whj=mp["Wh"][0:16, :],
                    wh399=jnp.concatenate([mp["Wh"][32:33, :]] * 3, axis=1),
                    wsi=wst[32:64, :], wsj=wst[0:32, :],
                    wsr=wst[64:80, :], wsv=wst[80:113, :],
                    bs=mp["bs"].reshape(1, -1), wu48=wu48,
                    wgt=mp["Wg"].T, bg=mp["bg"].reshape(1, -1))

    def ff_prep(fp):
        wst = fp["Ws"].T
        return dict(whf=fp["Wh"], wsfs=wst[0:32, :], wsfv=wst[32:48, :],
                    bsf=fp["bs"].reshape(1, -1), wuf=fp["Wu"],
                    wgft=fp["Wg"].T, bgf=fp["bg"].reshape(1, -1))

    m1 = msg_prep(p["layers"][0]["msg"])
    f1 = ff_prep(p["layers"][0]["ff"])
    m2 = msg_prep(p["layers"][1]["msg"])
    f2 = ff_prep(p["layers"][1]["ff"])
    go = p["gvp_out"]
    got = go["Ws"].T
    outwt = p["out_W"].T[:, :16]
    outb = p["out_b"][:16].reshape(1, 16)

    cparams = pltpu.CompilerParams(dimension_semantics=("parallel",))

    embed_ins = (t2, h, lig, xs,
                 p["lig_W"].T, p["lig_b"].reshape(1, -1),
                 p["h_W"].T, p["h_b"].reshape(1, -1),
                 p["node_in_W"].T, p["node_in_b"].reshape(1, -1),
                 m1["wsi"], m1["wsj"], m1["bs"])
    embed_specs = [
        _full(t2),
        pl.BlockSpec((B, h.shape[1]), lambda i: (i, 0)),
        pl.BlockSpec((B, lig.shape[1]), lambda i: (i, 0)),
        pl.BlockSpec((B, 4), lambda i: (i, 0)),
    ] + [_full(a) for a in embed_ins[4:]]
    w1_1, s0 = pl.pallas_call(
        _embed_body,
        grid=(NB,),
        in_specs=embed_specs,
        out_specs=[pl.BlockSpec((B, 68), lambda i: (i, 0)),
                   pl.BlockSpec((B, _HID), lambda i: (i, 0))],
        out_shape=[jax.ShapeDtypeStruct((NPAD, 68), _F32),
                   jax.ShapeDtypeStruct((NPAD, _HID), _F32)],
        compiler_params=cparams,
    )(*embed_ins)

    def layer_specs(ins):
        return ([_smem(), _smem()] + [_full(a) for a in ins[2:]])

    def msg_weight_ins(m, f):
        return (m["wh399"], m["wsr"], m["wsv"], m["bg"], m["wgt"], m["wu48"],
                f["whf"], f["wsfs"], f["wsfv"], f["bsf"], f["wuf"],
                f["wgft"], f["bgf"])

    l1_ins = ((lo_blk, hi_blk, w1_1, xr99, seld, sel33, sel48, s0)
              + msg_weight_ins(m1, f1)
              + (m2["wsi"], m2["wsj"], m2["bs"], m2["whi"], m2["whj"]))
    l1_out_specs = [
        pl.BlockSpec((B, 68), lambda i: (i, 0)),
        pl.BlockSpec((B, 99), lambda i: (i, 0)),
        pl.BlockSpec((B, 99), lambda i: (i, 0)),
        pl.BlockSpec((B, 80), lambda i: (i, 0)),
    ]
    l1_out_shape = [
        jax.ShapeDtypeStruct((NPAD, 68), _F32),
        jax.ShapeDtypeStruct((NPAD, 99), _F32),
        jax.ShapeDtypeStruct((NPAD, 99), _F32),
        jax.ShapeDtypeStruct((NPAD, 80), _F32),
    ]
    w1_2, w2_2, w3_2, w4_2 = pl.pallas_call(
        functools.partial(_layer_body, True, False, B),
        grid=(NB,),
        in_specs=layer_specs(l1_ins),
        out_specs=l1_out_specs,
        out_shape=l1_out_shape,
        compiler_params=cparams,
    )(*l1_ins)

    l2_ins = ((lo_blk, hi_blk, w1_2, xr99, seld, sel33, sel48,
               w2_2, w3_2, w4_2)
              + msg_weight_ins(m2, f2)
              + (go["Wh"], got[0:32, :], got[32:48, :],
                 go["bs"].reshape(1, -1), go["Wu"], go["Wg"].T,
                 go["bg"].reshape(1, 1), outwt, outb))
    out = pl.pallas_call(
        functools.partial(_layer_body, False, True, B),
        grid=(NB,),
        in_specs=layer_specs(l2_ins),
        out_specs=pl.BlockSpec((B, 19), lambda i: (i, 0)),
        out_shape=jax.ShapeDtypeStruct((NPAD, 19), _F32),
        compiler_params=cparams,
    )(*l2_ins)
    return out[:N]


# speedup vs baseline: 1.5194x; 1.5194x over previous
import functools

import jax
import jax.numpy as jnp
from jax import lax
from jax.experimental import pallas as pl
from jax.experimental.pallas import tpu as pltpu

_B = 24
_HID = 32
_NV = 16
_NRBF = 16
_DMAX = 10.0
_NORM = 100.0
_F32 = jnp.float32
_SEGPAD = 8000000.0


def _silu(x):
    return x * jax.nn.sigmoid(x)


def _dot(a, b):
    return jnp.dot(a, b, preferred_element_type=_F32)


def _embed_body(t_ref, h_ref, lig_ref, xs_ref, ligwt_ref, ligb_ref,
                hwt_ref, hb_ref, nodewt_ref, nodeb_ref,
                wsi1_ref, wsj1_ref, bs1_ref,
                w1_ref, s_ref):
    B = h_ref.shape[0]
    tcol = jnp.broadcast_to(t_ref[...], (B, 1))
    lig_in = jnp.concatenate([lig_ref[...], tcol], axis=1)
    lig_e = _silu(_dot(lig_in, ligwt_ref[...]) + ligb_ref[...]) * (1.0 / 0.6)
    h_e = _silu(_dot(h_ref[...], hwt_ref[...]) + hb_ref[...]) * (1.0 / 0.6)
    s0 = jnp.concatenate([h_e, lig_e], axis=1)
    s = _silu(_dot(s0, nodewt_ref[...]) + nodeb_ref[...])
    asi = _dot(s, wsi1_ref[...]) + bs1_ref[...]
    asj = _dot(s, wsj1_ref[...])
    w1_ref[...] = jnp.concatenate([asi, asj, xs_ref[...]], axis=1)
    s_ref[...] = s


def _layer_body(first, last, B, *args):
    if first:
        (lo_ref, hi_ref, w1_ref, xr_ref, seld_ref, sel33_ref,
         sel48_ref, s_ref, *rest) = args
        w2_ref = w3_ref = sv_ref = None
    else:
        (lo_ref, hi_ref, w1_ref, xr_ref, seld_ref, sel33_ref,
         sel48_ref, w2_ref, w3_ref, sv_ref, *rest) = args
    (wh399_ref, wsr_ref, wsv_ref, bg_ref, wgt_ref, wu48_ref,
     whf_ref, wsfs_ref, wsfv_ref, bsf_ref, wuf_ref, wgft_ref, bgf_ref,
     *tail) = rest
    if last:
        (who_ref, wsos_ref, wsov_ref, bso_ref, wuo_ref, wgot_ref, bgo_ref,
         outwt_ref, outb_ref, out_ref) = tail
    else:
        (wsi2_ref, wsj2_ref, bs2_ref, whi2_ref, whj2_ref,
         w1o_ref, w2o_ref, w3o_ref, w4o_ref) = tail

    i = pl.program_id(0)
    r0 = i * B
    P = B * B

    w1r = w1_ref[pl.ds(r0, B), :]
    asi3 = w1r[:, 0:32].reshape(B, 1, _HID)
    segr3 = w1r[:, 67:68].reshape(B, 1, 1)
    xr99 = xr_ref[pl.ds(r0, B), :].reshape(B, 1, 99)
    ri = lax.broadcasted_iota(jnp.int32, (B, 1, 1), 0) + r0
    if not first:
        aiwh3 = w2_ref[pl.ds(r0, B), :].reshape(B, 1, 99)

    wh399 = wh399_ref[...].reshape(1, 1, 99)
    wsr = wsr_ref[...]
    wsv = wsv_ref[...]
    bg = bg_ref[...]
    wgt = wgt_ref[...]
    wu48 = wu48_ref[...]
    seld = seld_ref[...]
    sel33 = sel33_ref[...]
    sel48 = sel48_ref[...]
    mu3 = (lax.broadcasted_iota(jnp.int32, (1, 1, _NRBF), 2).astype(_F32)
           * (_DMAX / (_NRBF - 1)))
    inv_sig = _NRBF / _DMAX

    def pair_block(j, carry):
        s_acc, v_acc = carry
        c0 = j * B
        w1c = w1_ref[pl.ds(c0, B), :]
        asj3 = w1c[:, 32:64].reshape(1, B, _HID)
        segc3 = w1c[:, 67:68].reshape(1, B, 1)
        rel99 = xr99 - xr_ref[pl.ds(c0, B), :].reshape(1, B, 99)
        d2p = _dot((rel99 * rel99).reshape(P, 99), seld)
        tp = d2p + 1e-8
        invp = lax.rsqrt(tp)
        d3 = (tp * invp).reshape(B, B, 1)
        invd3 = invp.reshape(B, B, 1)
        rbf = jnp.exp(-(((d3 - mu3) * inv_sig) ** 2))
        ci = lax.broadcasted_iota(jnp.int32, (1, B, 1), 1) + c0
        mask = jnp.logical_and(segr3 == segc3, ri != ci)
        mf1 = mask.astype(_F32).reshape(P, 1)
        if first:
            vh = (rel99 * invd3) * wh399
        else:
            vh = (aiwh3 + w3_ref[pl.ds(c0, B), :].reshape(1, B, 99)
                  + (rel99 * invd3) * wh399)
        sq = vh * vh
        vn = jnp.sqrt(_dot(sq.reshape(P, 99), sel33) + 1e-8)
        sm = ((asi3 + asj3).reshape(P, _HID)
              + _dot(rbf.reshape(P, _NRBF), wsr)
              + _dot(vn, wsv))
        sma = sm * jax.nn.sigmoid(sm)
        gate = jax.nn.sigmoid(_dot(sma, wgt) + bg)
        vu = _dot(vh.reshape(P, 99), wu48)
        s_add = (sma * mf1).reshape(B, B, _HID).sum(axis=1)
        gm48 = _dot(gate * mf1, sel48)
        v_add = (vu * gm48).reshape(B, B, 48).sum(axis=1)
        return (s_acc + s_add, v_acc + v_add)

    init = (jnp.zeros((B, _HID), _F32), jnp.zeros((B, 48), _F32))
    jlo = lo_ref[i]
    jhi = hi_ref[i]
    s_acc, v_acc = lax.fori_loop(jlo, jhi + 1, pair_block, init)

    if first:
        s_r = s_ref[pl.ds(r0, B), :]
        v1c = [v_acc[:, 16 * c:16 * c + 16] * (1.0 / _NORM) for c in range(3)]
    else:
        svr = sv_ref[pl.ds(r0, B), :]
        s_r = svr[:, 0:32]
        v1c = [svr[:, 32 + 16 * c:48 + 16 * c]
               + v_acc[:, 16 * c:16 * c + 16] * (1.0 / _NORM)
               for c in range(3)]
    s1 = s_r + s_acc * (1.0 / _NORM)

    vhf = [_dot(v1c[c], whf_ref[...]) for c in range(3)]
    vnf = jnp.sqrt(vhf[0] * vhf[0] + vhf[1] * vhf[1] + vhf[2] * vhf[2] + 1e-8)
    smf = _dot(s1, wsfs_ref[...]) + _dot(vnf, wsfv_ref[...]) + bsf_ref[...]
    smfa = smf * jax.nn.sigmoid(smf)
    gatef = jax.nn.sigmoid(_dot(smfa, wgft_ref[...]) + bgf_ref[...])
    s2 = s1 + smfa
    v2c = [v1c[c] + _dot(vhf[c], wuf_ref[...]) * gatef for c in range(3)]

    if not last:
        asi2 = _dot(s2, wsi2_ref[...]) + bs2_ref[...]
        asj2 = _dot(s2, wsj2_ref[...])
        w1o_ref[...] = jnp.concatenate([asi2, asj2, w1r[:, 64:68]], axis=1)
        w2o_ref[...] = jnp.concatenate(
            [_dot(v2c[c], whi2_ref[...]) for c in range(3)], axis=1)
        w3o_ref[...] = jnp.concatenate(
            [_dot(v2c[c], whj2_ref[...]) for c in range(3)], axis=1)
        w4o_ref[...] = jnp.concatenate([s2] + v2c, axis=1)
    else:
        vho = [_dot(v2c[c], who_ref[...]) for c in range(3)]
        vno = jnp.sqrt(vho[0] * vho[0] + vho[1] * vho[1] + vho[2] * vho[2]
                       + 1e-8)
        smo = _dot(s2, wsos_ref[...]) + _dot(vno, wsov_ref[...]) + bso_ref[...]
        smoa = smo * jax.nn.sigmoid(smo)
        gateo = jax.nn.sigmoid(_dot(smoa, wgot_ref[...]) + bgo_ref[...])
        vel = [_dot(vho[c], wuo_ref[...]) * gateo for c in range(3)]
        hf = _dot(smo, outwt_ref[...]) + outb_ref[...]
        out_ref[...] = jnp.concatenate(vel + [hf], axis=1)


def _full(a):
    nd = a.ndim
    return pl.BlockSpec(a.shape, lambda i, _nd=nd: (0,) * _nd)


def _smem():
    return pl.BlockSpec(memory_space=pltpu.SMEM)


def kernel(t, xh, batch_seg, ligand_site, params):
    N = xh.shape[0]
    B = _B
    NB = -(-N // B)
    NPAD = NB * B
    padn = NPAD - N

    x = jnp.pad(xh[:, :3].astype(_F32), ((0, padn), (0, 0)))
    h = jnp.pad(xh[:, 3:].astype(_F32), ((0, padn), (0, 0)))
    lig = jnp.pad(ligand_site.astype(_F32), ((0, padn), (0, 0)))
    seg = batch_seg.astype(jnp.int32)
    segp = jnp.pad(seg, (0, padn), constant_values=jnp.int32(int(_SEGPAD)))
    segf = segp.astype(_F32).reshape(NPAD, 1)
    xs = jnp.concatenate([x, segf], axis=1)
    xr99 = jnp.repeat(x, 33, axis=1)
    seld = jnp.zeros((99, 1), _F32).at[0, 0].set(1.0)
    seld = seld.at[33, 0].set(1.0).at[66, 0].set(1.0)
    sel33 = jnp.tile(jnp.eye(33, dtype=_F32), (3, 1))
    sel48 = jnp.tile(jnp.eye(16, dtype=_F32), (1, 3))
    blocks = segp.reshape(NB, B)
    lo_blk = (jnp.searchsorted(segp, blocks[:, 0], side="left") // B
              ).astype(jnp.int32)
    hi_blk = ((jnp.searchsorted(segp, blocks[:, -1], side="right") - 1) // B
              ).astype(jnp.int32)
    t2 = t.reshape(1, 1).astype(_F32)

    p = params

    def msg_prep(mp):
        wst = mp["Ws"].T
        wu = mp["Wu"]
        wu48 = jnp.zeros((99, 48), _F32)
        for c in range(3):
            wu48 = wu48.at[33 * c:33 * c + 33, 16 * c:16 * c + 16].set(wu)
        return dict(whi=mp["Wh"][16:32, :], whj=mp["Wh"][0:16, :],
                    wh399=jnp.concatenate([mp["Wh"][32:33, :]] * 3, axis=1),
                    wsi=wst[32:64, :], wsj=wst[0:32, :],
                    wsr=wst[64:80, :], wsv=wst[80:113, :],
                    bs=mp["bs"].reshape(1, -1), wu48=wu48,
                    wgt=mp["Wg"].T, bg=mp["bg"].reshape(1, -1))

    def ff_prep(fp):
        wst = fp["Ws"].T
        return dict(whf=fp["Wh"], wsfs=wst[0:32, :], wsfv=wst[32:48, :],
                    bsf=fp["bs"].reshape(1, -1), wuf=fp["Wu"],
                    wgft=fp["Wg"].T, bgf=fp["bg"].reshape(1, -1))

    m1 = msg_prep(p["layers"][0]["msg"])
    f1 = ff_prep(p["layers"][0]["ff"])
    m2 = msg_prep(p["layers"][1]["msg"])
    f2 = ff_prep(p["layers"][1]["ff"])
    go = p["gvp_out"]
    got = go["Ws"].T
    outwt = p["out_W"].T[:, :16]
    outb = p["out_b"][:16].reshape(1, 16)

    cparams = pltpu.CompilerParams(dimension_semantics=("parallel",))

    embed_ins = (t2, h, lig, xs,
                 p["lig_W"].T, p["lig_b"].reshape(1, -1),
                 p["h_W"].T, p["h_b"].reshape(1, -1),
                 p["node_in_W"].T, p["node_in_b"].reshape(1, -1),
                 m1["wsi"], m1["wsj"], m1["bs"])
    embed_specs = [
        _full(t2),
        pl.BlockSpec((B, h.shape[1]), lambda i: (i, 0)),
        pl.BlockSpec((B, lig.shape[1]), lambda i: (i, 0)),
        pl.BlockSpec((B, 4), lambda i: (i, 0)),
    ] + [_full(a) for a in embed_ins[4:]]
    w1_1, s0 = pl.pallas_call(
        _embed_body,
        grid=(NB,),
        in_specs=embed_specs,
        out_specs=[pl.BlockSpec((B, 68), lambda i: (i, 0)),
                   pl.BlockSpec((B, _HID), lambda i: (i, 0))],
        out_shape=[jax.ShapeDtypeStruct((NPAD, 68), _F32),
                   jax.ShapeDtypeStruct((NPAD, _HID), _F32)],
        compiler_params=cparams,
    )(*embed_ins)

    def layer_specs(ins):
        return ([_smem(), _smem()] + [_full(a) for a in ins[2:]])

    def msg_weight_ins(m, f):
        return (m["wh399"], m["wsr"], m["wsv"], m["bg"], m["wgt"], m["wu48"],
                f["whf"], f["wsfs"], f["wsfv"], f["bsf"], f["wuf"],
                f["wgft"], f["bgf"])

    l1_ins = ((lo_blk, hi_blk, w1_1, xr99, seld, sel33, sel48, s0)
              + msg_weight_ins(m1, f1)
              + (m2["wsi"], m2["wsj"], m2["bs"], m2["whi"], m2["whj"]))
    l1_out_specs = [
        pl.BlockSpec((B, 68), lambda i: (i, 0)),
        pl.BlockSpec((B, 99), lambda i: (i, 0)),
        pl.BlockSpec((B, 99), lambda i: (i, 0)),
        pl.BlockSpec((B, 80), lambda i: (i, 0)),
    ]
    l1_out_shape = [
        jax.ShapeDtypeStruct((NPAD, 68), _F32),
        jax.ShapeDtypeStruct((NPAD, 99), _F32),
        jax.ShapeDtypeStruct((NPAD, 99), _F32),
        jax.ShapeDtypeStruct((NPAD, 80), _F32),
    ]
    w1_2, w2_2, w3_2, w4_2 = pl.pallas_call(
        functools.partial(_layer_body, True, False, B),
        grid=(NB,),
        in_specs=layer_specs(l1_ins),
        out_specs=l1_out_specs,
        out_shape=l1_out_shape,
        compiler_params=cparams,
    )(*l1_ins)

    l2_ins = ((lo_blk, hi_blk, w1_2, xr99, seld, sel33, sel48,
               w2_2, w3_2, w4_2)
              + msg_weight_ins(m2, f2)
              + (go["Wh"], got[0:32, :], got[32:48, :],
                 go["bs"].reshape(1, -1), go["Wu"], go["Wg"].T,
                 go["bg"].reshape(1, 1), outwt, outb))
    out = pl.pallas_call(
        functools.partial(_layer_body, False, True, B),
        grid=(NB,),
        in_specs=layer_specs(l2_ins),
        out_specs=pl.BlockSpec((B, 19), lambda i: (i, 0)),
        out_shape=jax.ShapeDtypeStruct((NPAD, 19), _F32),
        compiler_params=cparams,
    )(*l2_ins)
    return out[:N]


# speedup vs baseline: 1.6782x; 1.1045x over previous
import functools

import jax
import jax.numpy as jnp
from jax import lax
from jax.experimental import pallas as pl
from jax.experimental.pallas import tpu as pltpu

_B = 32
_HID = 32
_NV = 16
_NRBF = 16
_DMAX = 10.0
_NORM = 100.0
_F32 = jnp.float32
_SEGPAD = 8000000.0


def _silu(x):
    return x * jax.nn.sigmoid(x)


def _dot(a, b):
    return jnp.dot(a, b, preferred_element_type=_F32)


def _embed_body(t_ref, h_ref, lig_ref, xs_ref, ligwt_ref, ligb_ref,
                hwt_ref, hb_ref, nodewt_ref, nodeb_ref,
                wsi1_ref, wsj1_ref, bs1_ref,
                w1_ref, s_ref):
    B = h_ref.shape[0]
    tcol = jnp.broadcast_to(t_ref[...], (B, 1))
    lig_in = jnp.concatenate([lig_ref[...], tcol], axis=1)
    lig_e = _silu(_dot(lig_in, ligwt_ref[...]) + ligb_ref[...]) * (1.0 / 0.6)
    h_e = _silu(_dot(h_ref[...], hwt_ref[...]) + hb_ref[...]) * (1.0 / 0.6)
    s0 = jnp.concatenate([h_e, lig_e], axis=1)
    s = _silu(_dot(s0, nodewt_ref[...]) + nodeb_ref[...])
    asi = _dot(s, wsi1_ref[...]) + bs1_ref[...]
    asj = _dot(s, wsj1_ref[...])
    w1_ref[...] = jnp.concatenate([asi, asj, xs_ref[...]], axis=1)
    s_ref[...] = s


def _layer_body(first, last, B, *args):
    if first:
        (lo_ref, hi_ref, w1_ref, xr_ref, seld_ref, sel33_ref,
         sel48_ref, s_ref, *rest) = args
        w2_ref = w3_ref = sv_ref = None
    else:
        (lo_ref, hi_ref, w1_ref, xr_ref, seld_ref, sel33_ref,
         sel48_ref, w2_ref, w3_ref, sv_ref, *rest) = args
    (wh399_ref, wsr_ref, wsv_ref, bg_ref, wgt_ref, wu48_ref,
     whf_ref, wsfs_ref, wsfv_ref, bsf_ref, wuf_ref, wgft_ref, bgf_ref,
     *tail) = rest
    if last:
        (who_ref, wsos_ref, wsov_ref, bso_ref, wuo_ref, wgot_ref, bgo_ref,
         outwt_ref, outb_ref, out_ref) = tail
    else:
        (wsi2_ref, wsj2_ref, bs2_ref, whi2_ref, whj2_ref,
         w1o_ref, w2o_ref, w3o_ref, w4o_ref) = tail

    i = pl.program_id(0)
    r0 = i * B
    P = B * B

    w1r = w1_ref[pl.ds(r0, B), :]
    asi3 = w1r[:, 0:32].reshape(B, 1, _HID)
    segr3 = w1r[:, 67:68].reshape(B, 1, 1)
    xr99 = xr_ref[pl.ds(r0, B), :].reshape(B, 1, 99)
    ri = lax.broadcasted_iota(jnp.int32, (B, 1, 1), 0) + r0
    if not first:
        aiwh3 = w2_ref[pl.ds(r0, B), :].reshape(B, 1, 99)

    wh399 = wh399_ref[...].reshape(1, 1, 99)
    wsr = wsr_ref[...]
    wsv = wsv_ref[...]
    bg = bg_ref[...]
    wgt = wgt_ref[...]
    wu48 = wu48_ref[...]
    seld = seld_ref[...]
    sel33 = sel33_ref[...]
    sel48 = sel48_ref[...]
    mu3 = (lax.broadcasted_iota(jnp.int32, (1, 1, _NRBF), 2).astype(_F32)
           * (_DMAX / (_NRBF - 1)))
    inv_sig = _NRBF / _DMAX

    def pair_block(j, carry):
        s_acc, v_acc = carry
        c0 = j * B
        w1c = w1_ref[pl.ds(c0, B), :]
        asj3 = w1c[:, 32:64].reshape(1, B, _HID)
        segc3 = w1c[:, 67:68].reshape(1, B, 1)
        rel99 = xr99 - xr_ref[pl.ds(c0, B), :].reshape(1, B, 99)
        d2p = _dot((rel99 * rel99).reshape(P, 99), seld)
        tp = d2p + 1e-8
        invp = lax.rsqrt(tp)
        d3 = (tp * invp).reshape(B, B, 1)
        invd3 = invp.reshape(B, B, 1)
        rbf = jnp.exp(-(((d3 - mu3) * inv_sig) ** 2))
        ci = lax.broadcasted_iota(jnp.int32, (1, B, 1), 1) + c0
        mask = jnp.logical_and(segr3 == segc3, ri != ci)
        mf1 = mask.astype(_F32).reshape(P, 1)
        if first:
            vh = (rel99 * invd3) * wh399
        else:
            vh = (aiwh3 + w3_ref[pl.ds(c0, B), :].reshape(1, B, 99)
                  + (rel99 * invd3) * wh399)
        sq = vh * vh
        vn = jnp.sqrt(_dot(sq.reshape(P, 99), sel33) + 1e-8)
        sm = ((asi3 + asj3).reshape(P, _HID)
              + _dot(rbf.reshape(P, _NRBF), wsr)
              + _dot(vn, wsv))
        sma = sm * jax.nn.sigmoid(sm)
        gate = jax.nn.sigmoid(_dot(sma, wgt) + bg)
        vu = _dot(vh.reshape(P, 99), wu48)
        s_add = (sma * mf1).reshape(B, B, _HID).sum(axis=1)
        gm48 = _dot(gate * mf1, sel48)
        v_add = (vu * gm48).reshape(B, B, 48).sum(axis=1)
        return (s_acc + s_add, v_acc + v_add)

    init = (jnp.zeros((B, _HID), _F32), jnp.zeros((B, 48), _F32))
    jlo = lo_ref[i]
    jhi = hi_ref[i]
    s_acc, v_acc = lax.fori_loop(jlo, jhi + 1, pair_block, init)

    if first:
        s_r = s_ref[pl.ds(r0, B), :]
        v1c = [v_acc[:, 16 * c:16 * c + 16] * (1.0 / _NORM) for c in range(3)]
    else:
        svr = sv_ref[pl.ds(r0, B), :]
        s_r = svr[:, 0:32]
        v1c = [svr[:, 32 + 16 * c:48 + 16 * c]
               + v_acc[:, 16 * c:16 * c + 16] * (1.0 / _NORM)
               for c in range(3)]
    s1 = s_r + s_acc * (1.0 / _NORM)

    vhf = [_dot(v1c[c], whf_ref[...]) for c in range(3)]
    vnf = jnp.sqrt(vhf[0] * vhf[0] + vhf[1] * vhf[1] + vhf[2] * vhf[2] + 1e-8)
    smf = _dot(s1, wsfs_ref[...]) + _dot(vnf, wsfv_ref[...]) + bsf_ref[...]
    smfa = smf * jax.nn.sigmoid(smf)
    gatef = jax.nn.sigmoid(_dot(smfa, wgft_ref[...]) + bgf_ref[...])
    s2 = s1 + smfa
    v2c = [v1c[c] + _dot(vhf[c], wuf_ref[...]) * gatef for c in range(3)]

    if not last:
        asi2 = _dot(s2, wsi2_ref[...]) + bs2_ref[...]
        asj2 = _dot(s2, wsj2_ref[...])
        w1o_ref[...] = jnp.concatenate([asi2, asj2, w1r[:, 64:68]], axis=1)
        w2o_ref[...] = jnp.concatenate(
            [_dot(v2c[c], whi2_ref[...]) for c in range(3)], axis=1)
        w3o_ref[...] = jnp.concatenate(
            [_dot(v2c[c], whj2_ref[...]) for c in range(3)], axis=1)
        w4o_ref[...] = jnp.concatenate([s2] + v2c, axis=1)
    else:
        vho = [_dot(v2c[c], who_ref[...]) for c in range(3)]
        vno = jnp.sqrt(vho[0] * vho[0] + vho[1] * vho[1] + vho[2] * vho[2]
                       + 1e-8)
        smo = _dot(s2, wsos_ref[...]) + _dot(vno, wsov_ref[...]) + bso_ref[...]
        smoa = smo * jax.nn.sigmoid(smo)
        gateo = jax.nn.sigmoid(_dot(smoa, wgot_ref[...]) + bgo_ref[...])
        vel = [_dot(vho[c], wuo_ref[...]) * gateo for c in range(3)]
        hf = _dot(smo, outwt_ref[...]) + outb_ref[...]
        out_ref[...] = jnp.concatenate(vel + [hf], axis=1)


def _full(a):
    nd = a.ndim
    return pl.BlockSpec(a.shape, lambda i, _nd=nd: (0,) * _nd)


def _smem():
    return pl.BlockSpec(memory_space=pltpu.SMEM)


def kernel(t, xh, batch_seg, ligand_site, params):
    N = xh.shape[0]
    B = _B
    NB = -(-N // B)
    NPAD = NB * B
    padn = NPAD - N

    x = jnp.pad(xh[:, :3].astype(_F32), ((0, padn), (0, 0)))
    h = jnp.pad(xh[:, 3:].astype(_F32), ((0, padn), (0, 0)))
    lig = jnp.pad(ligand_site.astype(_F32), ((0, padn), (0, 0)))
    seg = batch_seg.astype(jnp.int32)
    segp = jnp.pad(seg, (0, padn), constant_values=jnp.int32(int(_SEGPAD)))
    segf = segp.astype(_F32).reshape(NPAD, 1)
    xs = jnp.concatenate([x, segf], axis=1)
    xr99 = jnp.repeat(x, 33, axis=1)
    seld = jnp.zeros((99, 1), _F32).at[0, 0].set(1.0)
    seld = seld.at[33, 0].set(1.0).at[66, 0].set(1.0)
    sel33 = jnp.tile(jnp.eye(33, dtype=_F32), (3, 1))
    sel48 = jnp.tile(jnp.eye(16, dtype=_F32), (1, 3))
    blocks = segp.reshape(NB, B)
    lo_blk = (jnp.searchsorted(segp, blocks[:, 0], side="left") // B
              ).astype(jnp.int32)
    hi_blk = ((jnp.searchsorted(segp, blocks[:, -1], side="right") - 1) // B
              ).astype(jnp.int32)
    t2 = t.reshape(1, 1).astype(_F32)

    p = params

    def msg_prep(mp):
        wst = mp["Ws"].T
        wu = mp["Wu"]
        wu48 = jnp.zeros((99, 48), _F32)
        for c in range(3):
            wu48 = wu48.at[33 * c:33 * c + 33, 16 * c:16 * c + 16].set(wu)
        return dict(whi=mp["Wh"][16:32, :], whj=mp["Wh"][0:16, :],
                    wh399=jnp.concatenate([mp["Wh"][32:33, :]] * 3, axis=1),
                    wsi=wst[32:64, :], wsj=wst[0:32, :],
                    wsr=wst[64:80, :], wsv=wst[80:113, :],
                    bs=mp["bs"].reshape(1, -1), wu48=wu48,
                    wgt=mp["Wg"].T, bg=mp["bg"].reshape(1, -1))

    def ff_prep(fp):
        wst = fp["Ws"].T
        return dict(whf=fp["Wh"], wsfs=wst[0:32, :], wsfv=wst[32:48, :],
                    bsf=fp["bs"].reshape(1, -1), wuf=fp["Wu"],
                    wgft=fp["Wg"].T, bgf=fp["bg"].reshape(1, -1))

    m1 = msg_prep(p["layers"][0]["msg"])
    f1 = ff_prep(p["layers"][0]["ff"])
    m2 = msg_prep(p["layers"][1]["msg"])
    f2 = ff_prep(p["layers"][1]["ff"])
    go = p["gvp_out"]
    got = go["Ws"].T
    outwt = p["out_W"].T[:, :16]
    outb = p["out_b"][:16].reshape(1, 16)

    cparams = pltpu.CompilerParams(dimension_semantics=("parallel",))

    embed_ins = (t2, h, lig, xs,
                 p["lig_W"].T, p["lig_b"].reshape(1, -1),
                 p["h_W"].T, p["h_b"].reshape(1, -1),
                 p["node_in_W"].T, p["node_in_b"].reshape(1, -1),
                 m1["wsi"], m1["wsj"], m1["bs"])
    embed_specs = [
        _full(t2),
        pl.BlockSpec((B, h.shape[1]), lambda i: (i, 0)),
        pl.BlockSpec((B, lig.shape[1]), lambda i: (i, 0)),
        pl.BlockSpec((B, 4), lambda i: (i, 0)),
    ] + [_full(a) for a in embed_ins[4:]]
    w1_1, s0 = pl.pallas_call(
        _embed_body,
        grid=(NB,),
        in_specs=embed_specs,
        out_specs=[pl.BlockSpec((B, 68), lambda i: (i, 0)),
                   pl.BlockSpec((B, _HID), lambda i: (i, 0))],
        out_shape=[jax.ShapeDtypeStruct((NPAD, 68), _F32),
                   jax.ShapeDtypeStruct((NPAD, _HID), _F32)],
        compiler_params=cparams,
    )(*embed_ins)

    def layer_specs(ins):
        return ([_smem(), _smem()] + [_full(a) for a in ins[2:]])

    def msg_weight_ins(m, f):
        return (m["wh399"], m["wsr"], m["wsv"], m["bg"], m["wgt"], m["wu48"],
                f["whf"], f["wsfs"], f["wsfv"], f["bsf"], f["wuf"],
                f["wgft"], f["bgf"])

    l1_ins = ((lo_blk, hi_blk, w1_1, xr99, seld, sel33, sel48, s0)
              + msg_weight_ins(m1, f1)
              + (m2["wsi"], m2["wsj"], m2["bs"], m2["whi"], m2["whj"]))
    l1_out_specs = [
        pl.BlockSpec((B, 68), lambda i: (i, 0)),
        pl.BlockSpec((B, 99), lambda i: (i, 0)),
        pl.BlockSpec((B, 99), lambda i: (i, 0)),
        pl.BlockSpec((B, 80), lambda i: (i, 0)),
    ]
    l1_out_shape = [
        jax.ShapeDtypeStruct((NPAD, 68), _F32),
        jax.ShapeDtypeStruct((NPAD, 99), _F32),
        jax.ShapeDtypeStruct((NPAD, 99), _F32),
        jax.ShapeDtypeStruct((NPAD, 80), _F32),
    ]
    w1_2, w2_2, w3_2, w4_2 = pl.pallas_call(
        functools.partial(_layer_body, True, False, B),
        grid=(NB,),
        in_specs=layer_specs(l1_ins),
        out_specs=l1_out_specs,
        out_shape=l1_out_shape,
        compiler_params=cparams,
    )(*l1_ins)

    l2_ins = ((lo_blk, hi_blk, w1_2, xr99, seld, sel33, sel48,
               w2_2, w3_2, w4_2)
              + msg_weight_ins(m2, f2)
              + (go["Wh"], got[0:32, :], got[32:48, :],
                 go["bs"].reshape(1, -1), go["Wu"], go["Wg"].T,
                 go["bg"].reshape(1, 1), outwt, outb))
    out = pl.pallas_call(
        functools.partial(_layer_body, False, True, B),
        grid=(NB,),
        in_specs=layer_specs(l2_ins),
        out_specs=pl.BlockSpec((B, 19), lambda i: (i, 0)),
        out_shape=jax.ShapeDtypeStruct((NPAD, 19), _F32),
        compiler_params=cparams,
    )(*l2_ins)
    return out[:N]
